# baseline (speedup 1.0000x reference)
import functools

import jax
import jax.numpy as jnp
from jax import lax
from jax.experimental import pallas as pl
from jax.experimental.pallas import tpu as pltpu
from jax.experimental.pallas import tpu_sc as plsc

V = 1000000
D = 64
L = 50
M = 200
C = 100

S_LT = M + 1
S_ALL = S_LT + 1 + C
LP = 64
NC = 2
NS = 16
NW = NC * NS
NCHUNK = D // 16


def _encode_body(ids_hbm, lt_hbm, lt2_hbm, freqs_hbm, mems_out, ys_out,
                 idx_v, w_v, rows_v, acc_v, sem_r, sem_w):
  wid = lax.axis_index("s") * NC + lax.axis_index("c")
  nk = (S_ALL - wid + NW - 1) // NW

  def body(k, carry):
    seq = wid + k * NW
    pltpu.sync_copy(ids_hbm.at[seq], idx_v)
    cp_w = pltpu.async_copy(freqs_hbm.at[idx_v], w_v, sem_w)
    is_lt = seq < S_LT

    @pl.when(is_lt)
    def _():
      pltpu.async_copy(lt_hbm.at[idx_v.at[pl.ds(0, L)]], rows_v, sem_r).wait()

    @pl.when(jnp.logical_not(is_lt))
    def _():
      pltpu.async_copy(lt2_hbm.at[idx_v.at[pl.ds(0, L)]], rows_v, sem_r).wait()

    cp_w.wait()

    lanes = lax.iota(jnp.int32, 16)
    wsq = jnp.zeros((16,), jnp.float32)
    wchunks = []
    for c in range(NCHUNK):
      wv = w_v[pl.ds(c * 16, 16)]
      if (c + 1) * 16 > L:
        wv = jnp.where(lanes < (L - c * 16), wv, 0.0)
      wchunks.append(wv)
      wsq = wsq + wv * wv
    for sh in (8, 4, 2, 1):
      perm = lax.bitwise_xor(lanes, sh)
      wsq = wsq + jnp.take_along_axis(wsq, perm, axis=0)
    x = wsq

    i = lax.bitcast_convert_type(x, jnp.int32)
    i = 0x5F3759DF - lax.shift_right_logical(i, 1)
    y = lax.bitcast_convert_type(i, jnp.float32)
    for _ in range(3):
      y = y * (1.5 - 0.5 * x * y * y)

    accs = [jnp.zeros((16,), jnp.float32) for _ in range(NCHUNK)]
    for l in range(L):
      wl = jnp.take_along_axis(wchunks[l // 16],
                               jnp.full((16,), l % 16, jnp.int32), axis=0)
      for c in range(NCHUNK):
        accs[c] = accs[c] + wl * rows_v[l, pl.ds(c * 16, 16)]
    for c in range(NCHUNK):
      acc_v[pl.ds(c * 16, 16)] = accs[c] * y

    @pl.when(is_lt)
    def _():
      pltpu.sync_copy(acc_v, mems_out.at[seq])

    @pl.when(jnp.logical_not(is_lt))
    def _():
      pltpu.sync_copy(acc_v, ys_out.at[seq - S_LT])

    return carry

  lax.fori_loop(0, nk, body, 0)


@functools.cache
def _encode_sc():
  return functools.partial(
      pl.kernel,
      out_type=[
          jax.ShapeDtypeStruct((S_LT, D), jnp.float32),
          jax.ShapeDtypeStruct((1 + C, D), jnp.float32),
      ],
      mesh=plsc.VectorSubcoreMesh(core_axis_name="c", subcore_axis_name="s",
                                  num_cores=NC, num_subcores=NS),
      compiler_params=pltpu.CompilerParams(needs_layout_passes=False,
                                           use_tc_tiling_on_sc=False),
      scratch_types=[
          pltpu.VMEM((LP,), jnp.int32),
          pltpu.VMEM((LP,), jnp.float32),
          pltpu.VMEM((L, D), jnp.float32),
          pltpu.VMEM((D,), jnp.float32),
          pltpu.SemaphoreType.DMA,
          pltpu.SemaphoreType.DMA,
      ],
  )(_encode_body)


def _attn_body(m_ref, o_ref):
  m = m_ref[:]
  a = m[S_LT - 1:S_LT, :]
  nsq = jnp.sum(m * m, axis=1, keepdims=True)
  norms = jnp.sqrt(nsq)
  na = norms[S_LT - 1:S_LT, :]
  num = jnp.sum(m * a, axis=1, keepdims=True)
  den = jnp.maximum(na * norms, 1e-8)
  cos = num / den
  e = jnp.exp(cos - jnp.max(cos))
  p = e / jnp.sum(e)
  lhs = jnp.sum(p * m, axis=0, keepdims=True)
  o_ref[:] = jnp.broadcast_to(lhs, (1 + C, D))


_attn_tc = pl.pallas_call(
    _attn_body,
    out_shape=jax.ShapeDtypeStruct((1 + C, D), jnp.float32),
)


def kernel(xs, mems, ys, cands, lt, lt2, freqs):
  ids = jnp.concatenate([mems, xs, ys, cands], axis=0).astype(jnp.int32)
  ids = jnp.pad(ids, ((0, 0), (0, LP - L)))
  mems_enc, ys_cat = _encode_sc()(ids, lt, lt2, freqs)
  xs_cat = _attn_tc(mems_enc)
  return xs_cat, ys_cat


# speedup vs baseline: 1.5339x; 1.5339x over previous
import functools

import jax
import jax.numpy as jnp
from jax import lax
from jax.experimental import pallas as pl
from jax.experimental.pallas import tpu as pltpu
from jax.experimental.pallas import tpu_sc as plsc

V = 1000000
D = 64
L = 50
M = 200
C = 100

S_LT = M + 1
S_ALL = S_LT + 1 + C
LP = 128
NC = 2
NS = 16
NW = NC * NS
NCHUNK = D // 16


def _encode_body(idsf_hbm, lt_hbm, lt2_hbm, freqs_hbm,
                 mems_out, ys_out,
                 idx_v, w_v, rows8_v, acc_v, sem_r, sem_w):
  wid = lax.axis_index("s") * NC + lax.axis_index("c")
  nk = (S_ALL - wid + NW - 1) // NW

  def body(k, carry):
    seq = wid + k * NW
    base = seq * LP
    pltpu.sync_copy(idsf_hbm.at[pl.ds(base, LP)], idx_v)
    cp_w = pltpu.async_copy(freqs_hbm.at[idx_v.at[pl.ds(0, 64)]], w_v, sem_w)
    ichunks = [idx_v[pl.ds(c * 16, 16)] for c in range(NCHUNK)]
    starts = [pl.multiple_of(ichunks[l // 16][l % 16] & ~7, 8)
              for l in range(L)]
    is_lt = seq < S_LT

    @pl.when(is_lt)
    def _():
      for l in range(L):
        pltpu.async_copy(lt_hbm.at[pl.ds(starts[l], 8)], rows8_v.at[l], sem_r)

    @pl.when(jnp.logical_not(is_lt))
    def _():
      for l in range(L):
        pltpu.async_copy(lt2_hbm.at[pl.ds(starts[l], 8)], rows8_v.at[l], sem_r)

    cp_r = pltpu.make_async_copy(lt_hbm.at[pl.ds(0, 8)], rows8_v.at[0], sem_r)
    for _ in range(L):
      cp_r.wait()
    cp_w.wait()

    lanes = lax.iota(jnp.int32, 16)
    wsq = jnp.zeros((16,), jnp.float32)
    wchunks = []
    for c in range(NCHUNK):
      wv = w_v[pl.ds(c * 16, 16)]
      if (c + 1) * 16 > L:
        wv = jnp.where(lanes < (L - c * 16), wv, 0.0)
      wchunks.append(wv)
      wsq = wsq + wv * wv
    for sh in (8, 4, 2, 1):
      perm = lax.bitwise_xor(lanes, sh)
      wsq = wsq + jnp.take_along_axis(wsq, perm, axis=0)
    x = wsq

    i = lax.bitcast_convert_type(x, jnp.int32)
    i = 0x5F3759DF - lax.shift_right_logical(i, 1)
    y = lax.bitcast_convert_type(i, jnp.float32)
    for _ in range(3):
      y = y * (1.5 - 0.5 * x * y * y)

    accs = [jnp.zeros((16,), jnp.float32) for _ in range(NCHUNK)]
    for l in range(L):
      sub = ichunks[l // 16][l % 16] & 7
      wl = jnp.take_along_axis(wchunks[l // 16],
                               jnp.full((16,), l % 16, jnp.int32), axis=0)
      for c in range(NCHUNK):
        accs[c] = accs[c] + wl * rows8_v[l, sub, pl.ds(c * 16, 16)]
    for c in range(NCHUNK):
      acc_v[pl.ds(c * 16, 16)] = accs[c] * y

    @pl.when(is_lt)
    def _():
      pltpu.sync_copy(acc_v, mems_out.at[seq])

    @pl.when(jnp.logical_not(is_lt))
    def _():
      pltpu.sync_copy(acc_v, ys_out.at[seq - S_LT])

    return carry

  lax.fori_loop(0, nk, body, 0)


@functools.cache
def _encode_sc():
  return functools.partial(
      pl.kernel,
      out_type=[
          jax.ShapeDtypeStruct((S_LT, D), jnp.float32),
          jax.ShapeDtypeStruct((1 + C, D), jnp.float32),
      ],
      mesh=plsc.VectorSubcoreMesh(core_axis_name="c", subcore_axis_name="s",
                                  num_cores=NC, num_subcores=NS),
      compiler_params=pltpu.CompilerParams(needs_layout_passes=False),
      scratch_types=[
          pltpu.VMEM((LP,), jnp.int32),
          pltpu.VMEM((64,), jnp.float32),
          pltpu.VMEM((L, 8, D), jnp.float32),
          pltpu.VMEM((D,), jnp.float32),
          pltpu.SemaphoreType.DMA,
          pltpu.SemaphoreType.DMA,
      ],
  )(_encode_body)


def _attn_body(m_ref, o_ref):
  m = m_ref[:]
  a = m[S_LT - 1:S_LT, :]
  nsq = jnp.sum(m * m, axis=1, keepdims=True)
  norms = jnp.sqrt(nsq)
  na = norms[S_LT - 1:S_LT, :]
  num = jnp.sum(m * a, axis=1, keepdims=True)
  den = jnp.maximum(na * norms, 1e-8)
  cos = num / den
  e = jnp.exp(cos - jnp.max(cos))
  p = e / jnp.sum(e)
  lhs = jnp.sum(p * m, axis=0, keepdims=True)
  o_ref[:] = jnp.broadcast_to(lhs, (1 + C, D))


_attn_tc = pl.pallas_call(
    _attn_body,
    out_shape=jax.ShapeDtypeStruct((1 + C, D), jnp.float32),
)


def kernel(xs, mems, ys, cands, lt, lt2, freqs):
  ids = jnp.concatenate([mems, xs, ys, cands], axis=0).astype(jnp.int32)
  pad = (jnp.arange(S_ALL * LP, dtype=jnp.int32).reshape(S_ALL, LP) * 997) % V
  idsf = jnp.concatenate([ids, pad[:, L:]], axis=1).reshape(-1)
  mems_enc, ys_cat = _encode_sc()(idsf, lt, lt2, freqs)
  xs_cat = _attn_tc(mems_enc)
  return xs_cat, ys_cat


# speedup vs baseline: 4.7016x; 3.0651x over previous
import functools

import jax
import jax.numpy as jnp
from jax import lax
from jax.experimental import pallas as pl
from jax.experimental.pallas import tpu as pltpu
from jax.experimental.pallas import tpu_sc as plsc

V = 1000000
D = 64
L = 50
M = 200
C = 100

S_LT = M + 1
S_ALL = S_LT + 1 + C
LP = 128
NC = 2
NS = 16
NW = NC * NS
NCHUNK = D // 16


B = 4
NR = (L + B - 1) // B


def _encode_body(idsf_hbm, ltT_hbm, lt2T_hbm, freqs_hbm,
                 mems_out, ys_out,
                 idx_v, w_v, cols_v, acc_v, sem_r0, sem_r1, sem_w):
  wid = lax.axis_index("s") * NC + lax.axis_index("c")
  nk = (S_ALL - wid + NW - 1) // NW
  sems = (sem_r0, sem_r1)

  def body(k, carry):
    seq = wid + k * NW
    base = seq * LP
    pltpu.sync_copy(idsf_hbm.at[pl.ds(base, LP)], idx_v)
    cp_w = pltpu.async_copy(freqs_hbm.at[idx_v.at[pl.ds(0, 64)]], w_v, sem_w)
    ichunks = [idx_v[pl.ds(c * 16, 16)] for c in range(NCHUNK)]
    starts = [pl.multiple_of((ichunks[l // 16][l % 16] >> 7) << 7, 128)
              for l in range(L)]
    is_lt = seq < S_LT

    def issue(r):
      buf = r % 2
      ids_r = range(r * B, min((r + 1) * B, L))

      @pl.when(is_lt)
      def _():
        for b, l in enumerate(ids_r):
          pltpu.async_copy(ltT_hbm.at[:, pl.ds(starts[l], 128)],
                           cols_v.at[buf, b], sems[buf])

      @pl.when(jnp.logical_not(is_lt))
      def _():
        for b, l in enumerate(ids_r):
          pltpu.async_copy(lt2T_hbm.at[:, pl.ds(starts[l], 128)],
                           cols_v.at[buf, b], sems[buf])

    def drain(r):
      buf = r % 2
      n = min((r + 1) * B, L) - r * B
      cp = pltpu.make_async_copy(ltT_hbm.at[:, pl.ds(0, 128)],
                                 cols_v.at[buf, 0], sems[buf])
      for _ in range(n):
        cp.wait()

    issue(0)
    cp_w.wait()

    lanes = lax.iota(jnp.int32, 16)
    wsq = jnp.zeros((16,), jnp.float32)
    wchunks = []
    for c in range(NCHUNK):
      wv = w_v[pl.ds(c * 16, 16)]
      if (c + 1) * 16 > L:
        wv = jnp.where(lanes < (L - c * 16), wv, 0.0)
      wchunks.append(wv)
      wsq = wsq + wv * wv
    for sh in (8, 4, 2, 1):
      perm = lax.bitwise_xor(lanes, sh)
      wsq = wsq + jnp.take_along_axis(wsq, perm, axis=0)
    x = wsq

    i = lax.bitcast_convert_type(x, jnp.int32)
    i = 0x5F3759DF - lax.shift_right_logical(i, 1)
    y = lax.bitcast_convert_type(i, jnp.float32)
    for _ in range(3):
      y = y * (1.5 - 0.5 * x * y * y)

    accs = [jnp.zeros((16,), jnp.float32) for _ in range(NCHUNK)]
    dvs = [lax.iota(jnp.int32, 16) + (c * 16) for c in range(NCHUNK)]
    for r in range(NR):
      if r + 1 < NR:
        issue(r + 1)
      drain(r)
      buf = r % 2
      for b, l in enumerate(range(r * B, min((r + 1) * B, L))):
        sub = ichunks[l // 16][l % 16] & 127
        subv = jnp.full((16,), 0, jnp.int32) + sub
        bufv = jnp.full((16,), buf, jnp.int32)
        bv = jnp.full((16,), b, jnp.int32)
        wl = jnp.take_along_axis(wchunks[l // 16],
                                 jnp.full((16,), l % 16, jnp.int32), axis=0)
        for c in range(NCHUNK):
          g = plsc.load_gather(cols_v, [bufv, bv, dvs[c], subv])
          accs[c] = accs[c] + wl * g
    for c in range(NCHUNK):
      acc_v[pl.ds(c * 16, 16)] = accs[c] * y

    @pl.when(is_lt)
    def _():
      pltpu.sync_copy(acc_v, mems_out.at[seq])

    @pl.when(jnp.logical_not(is_lt))
    def _():
      pltpu.sync_copy(acc_v, ys_out.at[seq - S_LT])

    return carry

  lax.fori_loop(0, nk, body, 0)


@functools.cache
def _encode_sc():
  return functools.partial(
      pl.kernel,
      out_type=[
          jax.ShapeDtypeStruct((S_LT, D), jnp.float32),
          jax.ShapeDtypeStruct((1 + C, D), jnp.float32),
      ],
      mesh=plsc.VectorSubcoreMesh(core_axis_name="c", subcore_axis_name="s",
                                  num_cores=NC, num_subcores=NS),
      compiler_params=pltpu.CompilerParams(needs_layout_passes=False),
      scratch_types=[
          pltpu.VMEM((LP,), jnp.int32),
          pltpu.VMEM((64,), jnp.float32),
          pltpu.VMEM((2, B, D, 128), jnp.float32),
          pltpu.VMEM((D,), jnp.float32),
          pltpu.SemaphoreType.DMA,
          pltpu.SemaphoreType.DMA,
          pltpu.SemaphoreType.DMA,
      ],
  )(_encode_body)


def _attn_body(m_ref, o_ref):
  m = m_ref[:]
  a = m[S_LT - 1:S_LT, :]
  nsq = jnp.sum(m * m, axis=1, keepdims=True)
  norms = jnp.sqrt(nsq)
  na = norms[S_LT - 1:S_LT, :]
  num = jnp.sum(m * a, axis=1, keepdims=True)
  den = jnp.maximum(na * norms, 1e-8)
  cos = num / den
  e = jnp.exp(cos - jnp.max(cos))
  p = e / jnp.sum(e)
  lhs = jnp.sum(p * m, axis=0, keepdims=True)
  o_ref[:] = jnp.broadcast_to(lhs, (1 + C, D))


_attn_tc = pl.pallas_call(
    _attn_body,
    out_shape=jax.ShapeDtypeStruct((1 + C, D), jnp.float32),
)


def kernel(xs, mems, ys, cands, lt, lt2, freqs):
  ids = jnp.concatenate([mems, xs, ys, cands], axis=0).astype(jnp.int32)
  pad = (jnp.arange(S_ALL * LP, dtype=jnp.int32).reshape(S_ALL, LP) * 997) % V
  idsf = jnp.concatenate([ids, pad[:, L:]], axis=1).reshape(-1)
  mems_enc, ys_cat = _encode_sc()(idsf, lt.T, lt2.T, freqs)
  xs_cat = _attn_tc(mems_enc)
  return xs_cat, ys_cat


# speedup vs baseline: 4.7979x; 1.0205x over previous
import functools

import jax
import jax.numpy as jnp
from jax import lax
from jax.experimental import pallas as pl
from jax.experimental.pallas import tpu as pltpu
from jax.experimental.pallas import tpu_sc as plsc

V = 1000000
D = 64
L = 50
M = 200
C = 100

S_LT = M + 1
S_ALL = S_LT + 1 + C
LP = 128
NC = 2
NS = 16
NW = NC * NS
NCHUNK = D // 16


B = 6
NR = (L + B - 1) // B
KMAX = (S_ALL + NW - 1) // NW


def _encode_body(idsf_hbm, ltT_hbm, lt2T_hbm, freqs_hbm,
                 mems_out, ys_out,
                 idx_v, w_v, cols_v, acc_v, sem_r0, sem_r1, sem_w):
  wid = lax.axis_index("s") * NC + lax.axis_index("c")
  nk = (S_ALL - wid + NW - 1) // NW
  sems = (sem_r0, sem_r1)

  pltpu.sync_copy(idsf_hbm.at[pl.ds(wid * (KMAX * LP), KMAX * LP)], idx_v)

  def body(k, carry):
    seq = wid + k * NW
    base = k * LP
    cp_w = pltpu.async_copy(freqs_hbm.at[idx_v.at[pl.ds(base, 64)]], w_v,
                            sem_w)
    ichunks = [idx_v[pl.ds(base + c * 16, 16)] for c in range(NCHUNK)]
    starts = [pl.multiple_of((ichunks[l // 16][l % 16] >> 7) << 7, 128)
              for l in range(L)]
    is_lt = seq < S_LT

    def issue(r):
      buf = r % 2
      ids_r = range(r * B, min((r + 1) * B, L))

      @pl.when(is_lt)
      def _():
        for b, l in enumerate(ids_r):
          pltpu.async_copy(ltT_hbm.at[:, pl.ds(starts[l], 128)],
                           cols_v.at[buf, b], sems[buf])

      @pl.when(jnp.logical_not(is_lt))
      def _():
        for b, l in enumerate(ids_r):
          pltpu.async_copy(lt2T_hbm.at[:, pl.ds(starts[l], 128)],
                           cols_v.at[buf, b], sems[buf])

    def drain(r):
      buf = r % 2
      n = min((r + 1) * B, L) - r * B
      cp = pltpu.make_async_copy(ltT_hbm.at[:, pl.ds(0, 128)],
                                 cols_v.at[buf, 0], sems[buf])
      for _ in range(n):
        cp.wait()

    issue(0)
    cp_w.wait()

    lanes = lax.iota(jnp.int32, 16)
    wsq = jnp.zeros((16,), jnp.float32)
    wchunks = []
    for c in range(NCHUNK):
      wv = w_v[pl.ds(c * 16, 16)]
      if (c + 1) * 16 > L:
        wv = jnp.where(lanes < (L - c * 16), wv, 0.0)
      wchunks.append(wv)
      wsq = wsq + wv * wv
    for sh in (8, 4, 2, 1):
      perm = lax.bitwise_xor(lanes, sh)
      wsq = wsq + jnp.take_along_axis(wsq, perm, axis=0)
    x = wsq

    i = lax.bitcast_convert_type(x, jnp.int32)
    i = 0x5F3759DF - lax.shift_right_logical(i, 1)
    y = lax.bitcast_convert_type(i, jnp.float32)
    for _ in range(3):
      y = y * (1.5 - 0.5 * x * y * y)

    accs = [jnp.zeros((16,), jnp.float32) for _ in range(NCHUNK)]
    dvs = [lax.iota(jnp.int32, 16) + (c * 16) for c in range(NCHUNK)]
    for r in range(NR):
      if r + 1 < NR:
        issue(r + 1)
      drain(r)
      buf = r % 2
      for b, l in enumerate(range(r * B, min((r + 1) * B, L))):
        sub = ichunks[l // 16][l % 16] & 127
        subv = jnp.full((16,), 0, jnp.int32) + sub
        bufv = jnp.full((16,), buf, jnp.int32)
        bv = jnp.full((16,), b, jnp.int32)
        wl = jnp.take_along_axis(wchunks[l // 16],
                                 jnp.full((16,), l % 16, jnp.int32), axis=0)
        for c in range(NCHUNK):
          g = plsc.load_gather(cols_v, [bufv, bv, dvs[c], subv])
          accs[c] = accs[c] + wl * g
    for c in range(NCHUNK):
      acc_v[pl.ds(c * 16, 16)] = accs[c] * y

    @pl.when(is_lt)
    def _():
      pltpu.sync_copy(acc_v, mems_out.at[seq])

    @pl.when(jnp.logical_not(is_lt))
    def _():
      pltpu.sync_copy(acc_v, ys_out.at[seq - S_LT])

    return carry

  lax.fori_loop(0, nk, body, 0)


@functools.cache
def _encode_sc():
  return functools.partial(
      pl.kernel,
      out_type=[
          jax.ShapeDtypeStruct((S_LT, D), jnp.float32),
          jax.ShapeDtypeStruct((1 + C, D), jnp.float32),
      ],
      mesh=plsc.VectorSubcoreMesh(core_axis_name="c", subcore_axis_name="s",
                                  num_cores=NC, num_subcores=NS),
      compiler_params=pltpu.CompilerParams(needs_layout_passes=False),
      scratch_types=[
          pltpu.VMEM((KMAX * LP,), jnp.int32),
          pltpu.VMEM((64,), jnp.float32),
          pltpu.VMEM((2, B, D, 128), jnp.float32),
          pltpu.VMEM((D,), jnp.float32),
          pltpu.SemaphoreType.DMA,
          pltpu.SemaphoreType.DMA,
          pltpu.SemaphoreType.DMA,
      ],
  )(_encode_body)


def _attn_body(m_ref, o_ref):
  m = m_ref[:]
  a = m[S_LT - 1:S_LT, :]
  nsq = jnp.sum(m * m, axis=1, keepdims=True)
  norms = jnp.sqrt(nsq)
  na = norms[S_LT - 1:S_LT, :]
  num = jnp.sum(m * a, axis=1, keepdims=True)
  den = jnp.maximum(na * norms, 1e-8)
  cos = num / den
  e = jnp.exp(cos - jnp.max(cos))
  p = e / jnp.sum(e)
  lhs = jnp.sum(p * m, axis=0, keepdims=True)
  o_ref[:] = jnp.broadcast_to(lhs, (1 + C, D))


_attn_tc = pl.pallas_call(
    _attn_body,
    out_shape=jax.ShapeDtypeStruct((1 + C, D), jnp.float32),
)


def _worker_perm():
  perm = []
  for w in range(NW):
    for k in range(KMAX):
      seq = w + k * NW
      perm.append(seq if seq < S_ALL else S_ALL - 1)
  return jnp.array(perm, dtype=jnp.int32)


def kernel(xs, mems, ys, cands, lt, lt2, freqs):
  ids = jnp.concatenate([mems, xs, ys, cands], axis=0).astype(jnp.int32)
  pad = (jnp.arange(S_ALL * LP, dtype=jnp.int32).reshape(S_ALL, LP) * 997) % V
  idsp = jnp.concatenate([ids, pad[:, L:]], axis=1)
  idsf = idsp[_worker_perm()].reshape(-1)
  mems_enc, ys_cat = _encode_sc()(idsf, lt.T, lt2.T, freqs)
  xs_cat = _attn_tc(mems_enc)
  return xs_cat, ys_cat
